# Initial kernel scaffold; baseline (speedup 1.0000x reference)
import jax
import jax.numpy as jnp
from jax.experimental import pallas as pl


def kernel(x, edge_index, edge_attr, batch, W1, b1, W2, b2, W3, b3):
    raise NotImplementedError("write your pallas kernel here")


# speedup vs baseline: 17.9101x; 17.9101x over previous
import functools

import jax
import jax.numpy as jnp
from jax import lax
from jax.experimental import pallas as pl
from jax.experimental.pallas import tpu as pltpu
from jax.experimental.pallas import tpu_sc as plsc

NC = 2
NS = 16
NW = NC * NS
CH = 80
DEGW = 128
NG = 128


@functools.lru_cache(maxsize=None)
def _sc_deg(npad, e):
    ept = e // NW
    nch = ept // CH
    rows_pt = npad // NS
    mesh = plsc.VectorSubcoreMesh(core_axis_name="c", subcore_axis_name="s")

    @functools.partial(
        pl.kernel,
        mesh=mesh,
        out_type=jax.ShapeDtypeStruct((NC, npad, DEGW), jnp.float32),
        scratch_types=[
            pltpu.VMEM((nch, CH), jnp.int32),
            pltpu.VMEM((CH, DEGW), jnp.float32),
            pltpu.VMEM_SHARED((npad, DEGW), jnp.float32),
        ],
    )
    def k(dst_hbm, zeros_hbm, ones_hbm, out_hbm, idx_v, ones_v, acc):
        c = lax.axis_index("c")
        s = lax.axis_index("s")
        w = s * NC + c
        pltpu.sync_copy(dst_hbm.at[w], idx_v)
        pltpu.sync_copy(ones_hbm, ones_v)
        r0 = s * rows_pt
        pltpu.sync_copy(zeros_hbm.at[pl.ds(r0, rows_pt)],
                        acc.at[pl.ds(r0, rows_pt)])
        plsc.subcore_barrier()

        def chunk(j, carry):
            pltpu.sync_copy(ones_v, acc.at[idx_v.at[j]], add=True)
            return carry

        lax.fori_loop(0, nch, chunk, 0)
        plsc.subcore_barrier()
        pltpu.sync_copy(acc.at[pl.ds(r0, rows_pt)],
                        out_hbm.at[c, pl.ds(r0, rows_pt)])

    return k


@functools.lru_cache(maxsize=None)
def _sc_scatter(npad, e, h):
    ept = e // NW
    nch = ept // CH
    rows_pt = npad // NS
    mesh = plsc.VectorSubcoreMesh(core_axis_name="c", subcore_axis_name="s")

    @functools.partial(
        pl.kernel,
        mesh=mesh,
        out_type=jax.ShapeDtypeStruct((NC, npad, h), jnp.float32),
        scratch_types=[
            pltpu.VMEM((nch, CH), jnp.int32),
            pltpu.VMEM((nch, CH), jnp.int32),
            pltpu.VMEM((CH, h), jnp.float32),
            pltpu.VMEM_SHARED((npad, h), jnp.float32),
            pltpu.SemaphoreType.DMA,
        ],
    )
    def k(u_hbm, src_hbm, dst_hbm, zeros_hbm, out_hbm,
          src_v, dst_v, rows_v, acc, gsem):
        c = lax.axis_index("c")
        s = lax.axis_index("s")
        w = s * NC + c
        pltpu.sync_copy(src_hbm.at[w], src_v)
        pltpu.sync_copy(dst_hbm.at[w], dst_v)
        r0 = s * rows_pt
        pltpu.sync_copy(zeros_hbm.at[pl.ds(r0, rows_pt)],
                        acc.at[pl.ds(r0, rows_pt)])
        plsc.subcore_barrier()

        def chunk(j, carry):
            pltpu.async_copy(u_hbm.at[src_v.at[j]], rows_v, gsem).wait()
            pltpu.sync_copy(rows_v, acc.at[dst_v.at[j]], add=True)
            return carry

        lax.fori_loop(0, nch, chunk, 0)
        plsc.subcore_barrier()
        pltpu.sync_copy(acc.at[pl.ds(r0, rows_pt)],
                        out_hbm.at[c, pl.ds(r0, rows_pt)])

    return k


def _tc_prescale(p0, p1, x):
    n, d = x.shape
    r = 2528

    def body(p0b, p1b, xb, ub, db):
        deg = p0b[...][:, :1] + p1b[...][:, :1] + 1.0
        dinv = lax.rsqrt(deg)
        db[...] = dinv
        ub[...] = dinv * xb[...]

    return pl.pallas_call(
        body,
        grid=(n // r,),
        in_specs=[
            pl.BlockSpec((r, DEGW), lambda j: (j, 0)),
            pl.BlockSpec((r, DEGW), lambda j: (j, 0)),
            pl.BlockSpec((r, d), lambda j: (j, 0)),
        ],
        out_specs=[
            pl.BlockSpec((r, d), lambda j: (j, 0)),
            pl.BlockSpec((r, 1), lambda j: (j, 0)),
        ],
        out_shape=[
            jax.ShapeDtypeStruct((n, d), jnp.float32),
            jax.ShapeDtypeStruct((n, 1), jnp.float32),
        ],
    )(p0, p1, x)


def _tc_layer(s0, s1, u, dinv, w_mat, b_row):
    n, h = u.shape
    r = 2528

    def body(s0b, s1b, ub, db, wb, bb, ob):
        t = db[...] * (s0b[...] + s1b[...] + ub[...])
        z = jnp.dot(t, wb[...], preferred_element_type=jnp.float32) + bb[...]
        ob[...] = db[...] * jnp.maximum(z, 0.0)

    return pl.pallas_call(
        body,
        grid=(n // r,),
        in_specs=[
            pl.BlockSpec((r, h), lambda j: (j, 0)),
            pl.BlockSpec((r, h), lambda j: (j, 0)),
            pl.BlockSpec((r, h), lambda j: (j, 0)),
            pl.BlockSpec((r, 1), lambda j: (j, 0)),
            pl.BlockSpec((h, h), lambda j: (0, 0)),
            pl.BlockSpec((1, h), lambda j: (0, 0)),
        ],
        out_specs=pl.BlockSpec((r, h), lambda j: (j, 0)),
        out_shape=jax.ShapeDtypeStruct((n, h), jnp.float32),
    )(s0, s1, u, dinv, w_mat, b_row)


def _tc_final(s0, s1, u, dinv, w2, b2_row, batch_pad, w3, b3_row):
    n, h = u.shape
    out_d = w3.shape[1]
    r = 2528
    nsteps = n // r
    batch_3d = batch_pad.reshape(nsteps, 1, r)

    def body(s0b, s1b, ub, db, w2b, b2b, batb, w3b, b3b, ob, psum, pcnt):
        j = pl.program_id(0)

        @pl.when(j == 0)
        def _():
            psum[...] = jnp.zeros_like(psum)
            pcnt[...] = jnp.zeros_like(pcnt)

        t = db[...] * (s0b[...] + s1b[...] + ub[...])
        z = jnp.dot(t, w2b[...], preferred_element_type=jnp.float32) + b2b[...]
        h2 = jnp.maximum(z, 0.0)
        gid = lax.broadcasted_iota(jnp.int32, (NG, 1), 0)
        oh = (gid == batb[...].reshape(1, r)).astype(jnp.float32)
        psum[...] += jnp.dot(oh, h2, preferred_element_type=jnp.float32)
        pcnt[...] += jnp.sum(oh, axis=1, keepdims=True)

        @pl.when(j == nsteps - 1)
        def _():
            pooled = psum[...] / jnp.maximum(pcnt[...], 1.0)
            ob[...] = (jnp.dot(pooled, w3b[...],
                               preferred_element_type=jnp.float32) + b3b[...])

    return pl.pallas_call(
        body,
        grid=(nsteps,),
        in_specs=[
            pl.BlockSpec((r, h), lambda j: (j, 0)),
            pl.BlockSpec((r, h), lambda j: (j, 0)),
            pl.BlockSpec((r, h), lambda j: (j, 0)),
            pl.BlockSpec((r, 1), lambda j: (j, 0)),
            pl.BlockSpec((h, h), lambda j: (0, 0)),
            pl.BlockSpec((1, h), lambda j: (0, 0)),
            pl.BlockSpec((1, 1, r), lambda j: (j, 0, 0)),
            pl.BlockSpec((h, out_d), lambda j: (0, 0)),
            pl.BlockSpec((1, out_d), lambda j: (0, 0)),
        ],
        out_specs=pl.BlockSpec((NG, out_d), lambda j: (0, 0)),
        out_shape=jax.ShapeDtypeStruct((NG, out_d), jnp.float32),
        scratch_shapes=[
            pltpu.VMEM((NG, h), jnp.float32),
            pltpu.VMEM((NG, 1), jnp.float32),
        ],
    )(s0, s1, u, dinv, w2, b2_row, batch_3d, w3, b3_row)


def kernel(x, edge_index, edge_attr, batch, W1, b1, W2, b2, W3, b3):
    n, d = x.shape
    e = edge_index.shape[1]
    h = W1.shape[1]
    out_d = W3.shape[1]
    nch = e // (NW * CH)
    npad = ((n + 127) // 128) * 128

    src3 = edge_index[0].reshape(NW, nch, CH)
    dst3 = edge_index[1].reshape(NW, nch, CH)
    xp = jnp.pad(x, ((0, npad - n), (0, 0)))
    batch_pad = jnp.pad(batch.astype(jnp.int32), (0, npad - n),
                        constant_values=NG)
    zh = jnp.zeros((npad, h), jnp.float32)
    ones_d = jnp.ones((CH, DEGW), jnp.float32)

    degp = _sc_deg(npad, e)(dst3, jnp.zeros((npad, DEGW), jnp.float32), ones_d)
    u1, dinv = _tc_prescale(degp[0], degp[1], xp)

    sck = _sc_scatter(npad, e, h)
    s1 = sck(u1, src3, dst3, zh)
    u2 = _tc_layer(s1[0], s1[1], u1, dinv, W1, b1.reshape(1, h))
    s2 = sck(u2, src3, dst3, zh)
    return _tc_final(s2[0], s2[1], u2, dinv, W2, b2.reshape(1, h),
                     batch_pad, W3, b3.reshape(1, out_d))
